# baseline (speedup 1.0000x reference)
import functools

import jax
import jax.numpy as jnp
from jax import lax
from jax.experimental import pallas as pl
from jax.experimental.pallas import tpu as pltpu
from jax.experimental.pallas import tpu_sc as plsc

BATCH = 4096
SEQ = 200
EMB = 64
HIDDEN = 256
CLASSES = 16

NUM_CORES = 2
NUM_SUBCORES = 16
LANES = 16
NW = NUM_CORES * NUM_SUBCORES
ROWS_PER_W = BATCH // NW
NBUF = 2
SPLIT = 128
REST = SEQ - SPLIT

_mesh = plsc.VectorSubcoreMesh(
    core_axis_name="c", subcore_axis_name="s",
    num_cores=NUM_CORES, num_subcores=NUM_SUBCORES)


@functools.partial(
    pl.kernel,
    mesh=_mesh,
    compiler_params=pltpu.CompilerParams(use_tc_tiling_on_sc=False),
    out_type=jax.ShapeDtypeStruct((BATCH, EMB), jnp.float32),
    scratch_types=[
        pltpu.VMEM((ROWS_PER_W, SEQ), jnp.int32),
        pltpu.VMEM((NBUF, SEQ, EMB), jnp.float32),
        pltpu.VMEM((ROWS_PER_W, EMB), jnp.float32),
        pltpu.SemaphoreType.DMA,
        pltpu.SemaphoreType.DMA,
    ],
)
def _pool(x_hbm, table_hbm, out_hbm, idx_v, rows_v, pool_v, sem0, sem1):
    wid = lax.axis_index("s") * NUM_CORES + lax.axis_index("c")
    base = wid * ROWS_PER_W
    sems = [sem0, sem1]

    pltpu.sync_copy(x_hbm.at[pl.ds(base, ROWS_PER_W)], idx_v)

    def issue(r, slot):
        pltpu.make_async_copy(
            table_hbm.at[idx_v.at[r, pl.ds(0, SPLIT)]],
            rows_v.at[slot, pl.ds(0, SPLIT)],
            sems[slot]).start()
        pltpu.make_async_copy(
            table_hbm.at[idx_v.at[r, pl.ds(SPLIT, REST)]],
            rows_v.at[slot, pl.ds(SPLIT, REST)],
            sems[slot]).start()

    def wait_slot(slot):
        pltpu.make_async_copy(
            table_hbm.at[pl.ds(0, SEQ)], rows_v.at[slot], sems[slot]).wait()

    def reduce_row(slot, r):
        def body(i, accs):
            return tuple(accs[c] + rows_v[slot, i, pl.ds(LANES * c, LANES)]
                         for c in range(EMB // LANES))
        zero = jnp.zeros((LANES,), jnp.float32)
        accs = lax.fori_loop(0, SEQ, body, (zero,) * (EMB // LANES),
                             unroll=8)
        for c in range(EMB // LANES):
            pool_v[r, pl.ds(LANES * c, LANES)] = accs[c] * (1.0 / SEQ)

    issue(0, 0)

    def outer(g, _):
        for b in range(NBUF):
            r = g * NBUF + b
            nxt = r + 1

            @pl.when(nxt < ROWS_PER_W)
            def _():
                issue(nxt, (b + 1) % NBUF)

            wait_slot(b)
            reduce_row(b, r)
        return 0

    lax.fori_loop(0, ROWS_PER_W // NBUF, outer, 0)
    pltpu.sync_copy(pool_v, out_hbm.at[pl.ds(base, ROWS_PER_W)])


def _mlp_body(x_ref, w1_ref, b1_ref, w2_ref, b2_ref, o_ref):
    h = jnp.dot(x_ref[...], w1_ref[...],
                preferred_element_type=jnp.float32) + b1_ref[...]
    logits = jnp.dot(h, w2_ref[...],
                     preferred_element_type=jnp.float32) + b2_ref[...]
    m = jnp.max(logits, axis=-1, keepdims=True)
    e = jnp.exp(logits - m)
    o_ref[...] = e / jnp.sum(e, axis=-1, keepdims=True)


_mlp = pl.pallas_call(
    _mlp_body,
    out_shape=jax.ShapeDtypeStruct((BATCH, CLASSES), jnp.float32),
)


def kernel(x, emb_table, W1, b1, W2, b2):
    pooled = _pool(x.astype(jnp.int32), emb_table)
    return _mlp(pooled, W1, b1.reshape(1, HIDDEN), W2, b2.reshape(1, CLASSES))
